# baseline (device time: 85810 ns/iter reference)
import jax
import jax.numpy as jnp
from jax import lax
from jax.experimental import pallas as pl
from jax.experimental.pallas import tpu as pltpu

N_DEV = 4


def kernel(x, w_mat):
    m, _ = x.shape
    _, n = w_mat.shape
    chunk = m // N_DEV

    def body(x_ref, w_ref, out_ref, comm_ref, send_sems, recv_sems):
        my_pos = lax.axis_index("i")
        left = lax.rem(my_pos + N_DEV - 1, N_DEV)
        right = lax.rem(my_pos + 1, N_DEV)

        barrier_sem = pltpu.get_barrier_semaphore()
        for nbr in (left, right):
            pl.semaphore_signal(
                barrier_sem, inc=1,
                device_id=(nbr,), device_id_type=pl.DeviceIdType.MESH,
            )
        pl.semaphore_wait(barrier_sem, 2)

        out_ref[:, :] = jnp.dot(
            x_ref[:, :], w_ref[:, :], preferred_element_type=jnp.float32
        )

        for h in range(N_DEV - 1):
            s = h % 3
            c_send = lax.rem(my_pos - h + 2 * N_DEV, N_DEV)
            c_recv = lax.rem(my_pos - h - 1 + 2 * N_DEV, N_DEV)
            rdma = pltpu.make_async_remote_copy(
                src_ref=out_ref.at[pl.ds(c_send * chunk, chunk), :],
                dst_ref=comm_ref.at[s],
                send_sem=send_sems.at[s],
                recv_sem=recv_sems.at[s],
                device_id=(right,),
                device_id_type=pl.DeviceIdType.MESH,
            )
            rdma.start()
            rdma.wait()
            out_ref[pl.ds(c_recv * chunk, chunk), :] = (
                out_ref[pl.ds(c_recv * chunk, chunk), :] + comm_ref[s, :, :]
            )

        g = lax.rem(my_pos + 1, N_DEV)
        out_ref[pl.ds(g * chunk, chunk), :] = jnp.maximum(
            out_ref[pl.ds(g * chunk, chunk), :], 0.0
        )

        for h in range(N_DEV - 1):
            s = h % 3
            c_send = lax.rem(g - h + 2 * N_DEV, N_DEV)
            rdma = pltpu.make_async_remote_copy(
                src_ref=out_ref.at[pl.ds(c_send * chunk, chunk), :],
                dst_ref=out_ref.at[pl.ds(c_send * chunk, chunk), :],
                send_sem=send_sems.at[s],
                recv_sem=recv_sems.at[s],
                device_id=(right,),
                device_id_type=pl.DeviceIdType.MESH,
            )
            rdma.start()
            rdma.wait()

    return pl.pallas_call(
        body,
        out_shape=jax.ShapeDtypeStruct((m, n), jnp.float32),
        in_specs=[
            pl.BlockSpec(memory_space=pltpu.VMEM),
            pl.BlockSpec(memory_space=pltpu.VMEM),
        ],
        out_specs=pl.BlockSpec(memory_space=pltpu.VMEM),
        scratch_shapes=[
            pltpu.VMEM((3, chunk, n), jnp.float32),
            pltpu.SemaphoreType.DMA((3,)),
            pltpu.SemaphoreType.DMA((3,)),
        ],
        compiler_params=pltpu.CompilerParams(collective_id=0),
    )(x, w_mat)


# device time: 52230 ns/iter; 1.6429x vs baseline; 1.6429x over previous
import jax
import jax.numpy as jnp
from jax import lax
from jax.experimental import pallas as pl
from jax.experimental.pallas import tpu as pltpu

N_DEV = 4


def kernel(x, w_mat):
    m, _ = x.shape
    _, n = w_mat.shape
    chunk = m // N_DEV
    half = n // 2

    def body(x_ref, w_ref, out_ref, comm_cw, comm_ccw,
             send_cw, recv_cw, send_ccw, recv_ccw):
        my_pos = lax.axis_index("i")
        left = lax.rem(my_pos + N_DEV - 1, N_DEV)
        right = lax.rem(my_pos + 1, N_DEV)

        barrier_sem = pltpu.get_barrier_semaphore()
        for nbr in (left, right):
            pl.semaphore_signal(
                barrier_sem, inc=1,
                device_id=(nbr,), device_id_type=pl.DeviceIdType.MESH,
            )
        pl.semaphore_wait(barrier_sem, 2)

        out_ref[:, :] = jnp.dot(
            x_ref[:, :], w_ref[:, :], preferred_element_type=jnp.float32
        )

        def rows(c):
            return pl.ds(c * chunk, chunk)

        cw_cols = pl.ds(0, half)
        ccw_cols = pl.ds(half, half)

        for h in range(N_DEV - 1):
            s = h % 3
            c_s_cw = lax.rem(my_pos - h + 2 * N_DEV, N_DEV)
            c_r_cw = lax.rem(my_pos - h - 1 + 2 * N_DEV, N_DEV)
            c_s_ccw = lax.rem(my_pos + h, N_DEV)
            c_r_ccw = lax.rem(my_pos + h + 1, N_DEV)
            rdma_cw = pltpu.make_async_remote_copy(
                src_ref=out_ref.at[rows(c_s_cw), cw_cols],
                dst_ref=comm_cw.at[s],
                send_sem=send_cw.at[s],
                recv_sem=recv_cw.at[s],
                device_id=(right,),
                device_id_type=pl.DeviceIdType.MESH,
            )
            rdma_ccw = pltpu.make_async_remote_copy(
                src_ref=out_ref.at[rows(c_s_ccw), ccw_cols],
                dst_ref=comm_ccw.at[s],
                send_sem=send_ccw.at[s],
                recv_sem=recv_ccw.at[s],
                device_id=(left,),
                device_id_type=pl.DeviceIdType.MESH,
            )
            rdma_cw.start()
            rdma_ccw.start()
            rdma_cw.wait()
            rdma_ccw.wait()
            out_ref[rows(c_r_cw), cw_cols] = (
                out_ref[rows(c_r_cw), cw_cols] + comm_cw[s, :, :]
            )
            out_ref[rows(c_r_ccw), ccw_cols] = (
                out_ref[rows(c_r_ccw), ccw_cols] + comm_ccw[s, :, :]
            )

        g_cw = lax.rem(my_pos + 1, N_DEV)
        g_ccw = lax.rem(my_pos + N_DEV - 1, N_DEV)
        out_ref[rows(g_cw), cw_cols] = jnp.maximum(
            out_ref[rows(g_cw), cw_cols], 0.0
        )
        out_ref[rows(g_ccw), ccw_cols] = jnp.maximum(
            out_ref[rows(g_ccw), ccw_cols], 0.0
        )

        for h in range(N_DEV - 1):
            s = h % 3
            c_s_cw = lax.rem(g_cw - h + 2 * N_DEV, N_DEV)
            c_s_ccw = lax.rem(g_ccw + h, N_DEV)
            rdma_cw = pltpu.make_async_remote_copy(
                src_ref=out_ref.at[rows(c_s_cw), cw_cols],
                dst_ref=out_ref.at[rows(c_s_cw), cw_cols],
                send_sem=send_cw.at[s],
                recv_sem=recv_cw.at[s],
                device_id=(right,),
                device_id_type=pl.DeviceIdType.MESH,
            )
            rdma_ccw = pltpu.make_async_remote_copy(
                src_ref=out_ref.at[rows(c_s_ccw), ccw_cols],
                dst_ref=out_ref.at[rows(c_s_ccw), ccw_cols],
                send_sem=send_ccw.at[s],
                recv_sem=recv_ccw.at[s],
                device_id=(left,),
                device_id_type=pl.DeviceIdType.MESH,
            )
            rdma_cw.start()
            rdma_ccw.start()
            rdma_cw.wait()
            rdma_ccw.wait()

    return pl.pallas_call(
        body,
        out_shape=jax.ShapeDtypeStruct((m, n), jnp.float32),
        in_specs=[
            pl.BlockSpec(memory_space=pltpu.VMEM),
            pl.BlockSpec(memory_space=pltpu.VMEM),
        ],
        out_specs=pl.BlockSpec(memory_space=pltpu.VMEM),
        scratch_shapes=[
            pltpu.VMEM((3, chunk, half), jnp.float32),
            pltpu.VMEM((3, chunk, half), jnp.float32),
            pltpu.SemaphoreType.DMA((3,)),
            pltpu.SemaphoreType.DMA((3,)),
            pltpu.SemaphoreType.DMA((3,)),
            pltpu.SemaphoreType.DMA((3,)),
        ],
        compiler_params=pltpu.CompilerParams(collective_id=0),
    )(x, w_mat)


# device time: 44536 ns/iter; 1.9268x vs baseline; 1.1728x over previous
import jax
import jax.numpy as jnp
from jax import lax
from jax.experimental import pallas as pl
from jax.experimental.pallas import tpu as pltpu

N_DEV = 4
N_HOP = N_DEV - 1


def kernel(x, w_mat):
    m, _ = x.shape
    _, n = w_mat.shape
    chunk = m // N_DEV
    qw = n // 4

    quarters = ((0, 1), (1, 1), (2, -1), (3, -1))

    def body(x_ref, w_ref, out_ref,
             comm0, comm1, comm2, comm3,
             snd0, rcv0, snd1, rcv1, snd2, rcv2, snd3, rcv3):
        comm = (comm0, comm1, comm2, comm3)
        snd = (snd0, snd1, snd2, snd3)
        rcv = (rcv0, rcv1, rcv2, rcv3)

        my_pos = lax.axis_index("i")
        left = lax.rem(my_pos + N_DEV - 1, N_DEV)
        right = lax.rem(my_pos + 1, N_DEV)

        def cmod(e):
            return lax.rem(e + 4 * N_DEV, N_DEV)

        def rows(c):
            return pl.ds(c * chunk, chunk)

        def cols(q):
            return pl.ds(q * qw, qw)

        def mm_chunk(c):
            out_ref[rows(c), :] = jnp.dot(
                x_ref[rows(c), :], w_ref[:, :],
                preferred_element_type=jnp.float32,
            )

        def rs_rdma(q, d, h):
            s = h % 3
            c = cmod(my_pos - d * h)
            return pltpu.make_async_remote_copy(
                src_ref=out_ref.at[rows(c), cols(q)],
                dst_ref=comm[q].at[s],
                send_sem=snd[q].at[s],
                recv_sem=rcv[q].at[s],
                device_id=(right if d > 0 else left,),
                device_id_type=pl.DeviceIdType.MESH,
            )

        def ag_rdma(q, d, h):
            s = h % 3
            c = cmod(my_pos + d - d * h)
            ref = out_ref.at[rows(c), cols(q)]
            return pltpu.make_async_remote_copy(
                src_ref=ref,
                dst_ref=ref,
                send_sem=snd[q].at[s],
                recv_sem=rcv[q].at[s],
                device_id=(right if d > 0 else left,),
                device_id_type=pl.DeviceIdType.MESH,
            )

        barrier_sem = pltpu.get_barrier_semaphore()
        for nbr in (left, right):
            pl.semaphore_signal(
                barrier_sem, inc=1,
                device_id=(nbr,), device_id_type=pl.DeviceIdType.MESH,
            )
        mm_chunk(my_pos)
        pl.semaphore_wait(barrier_sem, 2)

        rs = {}
        for q, d in quarters:
            rs[(q, 0)] = rs_rdma(q, d, 0)
            rs[(q, 0)].start()
        for off in (-1, 1, 2):
            mm_chunk(cmod(my_pos + off))

        ag = {}
        for h in range(N_HOP):
            for q, d in quarters:
                rs[(q, h)].wait()
                c_r = cmod(my_pos - d * (h + 1))
                out_ref[rows(c_r), cols(q)] = (
                    out_ref[rows(c_r), cols(q)] + comm[q][h % 3, :, :]
                )
                if h + 1 < N_HOP:
                    rs[(q, h + 1)] = rs_rdma(q, d, h + 1)
                    rs[(q, h + 1)].start()
                else:
                    g = cmod(my_pos + d)
                    out_ref[rows(g), cols(q)] = jnp.maximum(
                        out_ref[rows(g), cols(q)], 0.0
                    )
                    ag[(q, 0)] = ag_rdma(q, d, 0)
                    ag[(q, 0)].start()

        for h in range(N_HOP):
            for q, d in quarters:
                ag[(q, h)].wait()
                if h + 1 < N_HOP:
                    ag[(q, h + 1)] = ag_rdma(q, d, h + 1)
                    ag[(q, h + 1)].start()

    return pl.pallas_call(
        body,
        out_shape=jax.ShapeDtypeStruct((m, n), jnp.float32),
        in_specs=[
            pl.BlockSpec(memory_space=pltpu.VMEM),
            pl.BlockSpec(memory_space=pltpu.VMEM),
        ],
        out_specs=pl.BlockSpec(memory_space=pltpu.VMEM),
        scratch_shapes=(
            [pltpu.VMEM((3, chunk, qw), jnp.float32)] * 4
            + [pltpu.SemaphoreType.DMA((3,))] * 8
        ),
        compiler_params=pltpu.CompilerParams(collective_id=0),
    )(x, w_mat)
